# baseline (speedup 1.0000x reference)
import functools

import jax
import jax.numpy as jnp
from jax import lax
from jax.experimental import pallas as pl
from jax.experimental.pallas import tpu as pltpu
from jax.experimental.pallas import tpu_sc as plsc

EMB = 64
ADIM = 13
NW = 32
SUB = 128
CHUNK = 512
NSUB = CHUNK // SUB


def _sc_embed(idx2d, audio, table, W, b, n_rows):
    rows_per_w = n_rows // NW
    n_chunks = rows_per_w // CHUNK
    mesh = plsc.VectorSubcoreMesh(core_axis_name="c", subcore_axis_name="s")

    @functools.partial(
        pl.kernel,
        mesh=mesh,
        compiler_params=pltpu.CompilerParams(use_tc_tiling_on_sc=False),
        out_type=jax.ShapeDtypeStruct((n_rows, EMB), jnp.float32),
        scratch_types=[
            pltpu.VMEM((NSUB, SUB), jnp.int32),
            pltpu.VMEM((CHUNK * ADIM + 16,), jnp.float32),
            pltpu.VMEM((CHUNK, EMB), jnp.float32),
            pltpu.VMEM((ADIM, EMB), jnp.float32),
            pltpu.VMEM((EMB,), jnp.float32),
            pltpu.SemaphoreType.DMA,
        ],
    )
    def k(idx_hbm, audio_hbm, table_hbm, w_hbm, b_hbm, out_hbm,
          idx_v, audio_v, rows_v, w_v, b_v, sem):
        wid = lax.axis_index("s") * 2 + lax.axis_index("c")
        base0 = wid * rows_per_w
        ib0 = wid * (rows_per_w // SUB)

        pltpu.sync_copy(w_hbm, w_v)
        pltpu.sync_copy(b_hbm, b_v)
        wvec = [[w_v[d, pl.ds(16 * q, 16)] for q in range(4)]
                for d in range(ADIM)]
        bvec = [b_v[pl.ds(16 * q, 16)] for q in range(4)]

        def chunk_body(ch, carry):
            base = base0 + ch * CHUNK
            ib = ib0 + ch * NSUB
            pltpu.sync_copy(idx_hbm.at[pl.ds(ib, NSUB)], idx_v)
            pltpu.sync_copy(audio_hbm.at[pl.ds(base * ADIM, CHUNK * ADIM)],
                            audio_v.at[pl.ds(0, CHUNK * ADIM)])
            cps = [
                pltpu.async_copy(
                    table_hbm.at[idx_v.at[j]],
                    rows_v.at[pl.ds(j * SUB, SUB)], sem)
                for j in range(NSUB)
            ]
            for cp in cps:
                cp.wait()

            for p in range(2):
                qs = (2 * p, 2 * p + 1)

                def row_body(r, c, qs=qs):
                    av = audio_v[pl.ds(r * ADIM, 16)]
                    a = [av[d] for d in range(ADIM)]
                    for q in qs:
                        col = pl.ds(16 * q, 16)
                        acc = rows_v[r, col] + bvec[q]
                        for d in range(ADIM):
                            acc = acc + a[d] * wvec[d][q]
                        rows_v[r, col] = acc
                    return c

                lax.fori_loop(0, CHUNK, row_body, 0)

            pltpu.sync_copy(rows_v, out_hbm.at[pl.ds(base, CHUNK)])
            return carry

        lax.fori_loop(0, n_chunks, chunk_body, 0)

    return k(idx2d, audio, table, W, b)


def kernel(o_idxs, audio_features, table, W, b):
    bsz, seq = o_idxs.shape
    n_rows = bsz * seq
    idx2d = o_idxs.reshape(n_rows // SUB, SUB).astype(jnp.int32)
    audio = audio_features.reshape(n_rows * ADIM).astype(jnp.float32)
    out = _sc_embed(idx2d, audio, table, W, b, n_rows)
    return out.reshape(bsz, seq, EMB)
